# baseline (speedup 1.0000x reference)
import functools

import jax
import jax.numpy as jnp
from jax import lax
from jax.experimental import pallas as pl
from jax.experimental.pallas import tpu as pltpu
from jax.experimental.pallas import tpu_sc as plsc

_V = 1000
_VPAD = 1024
_CHUNK = 32
_L = 16

_mesh = plsc.VectorSubcoreMesh(core_axis_name="c", subcore_axis_name="s")


@functools.partial(
    pl.kernel,
    mesh=_mesh,
    out_type=jax.ShapeDtypeStruct((_VPAD,), jnp.float32),
    scratch_types=[
        pltpu.VMEM((_L,), jnp.int32),
        pltpu.VMEM((_CHUNK,), jnp.float32),
    ],
    compiler_params=pltpu.CompilerParams(needs_layout_passes=False),
)
def _bow_sc(tok_hbm, out_hbm, tok_v, buf_v):
    wid = lax.axis_index("s") * 2 + lax.axis_index("c")
    base = wid * _CHUNK

    pltpu.sync_copy(tok_hbm, tok_v)
    toks = tok_v[...]

    zeros = jnp.zeros((_L,), jnp.float32)
    buf_v[pl.ds(0, _L)] = zeros
    buf_v[pl.ds(_L, _L)] = zeros

    lane = lax.iota(jnp.int32, _L)
    loc = toks - base
    in_range = (loc >= 0) & (loc < _CHUNK)
    loc_c = jnp.clip(loc, 0, _CHUNK - 1)
    ones = jnp.ones((_L,), jnp.float32)

    plsc.addupdate_scatter(buf_v, [loc_c], ones, mask=(lane == 0) & in_range)
    plsc.addupdate_scatter(buf_v, [loc_c], ones, mask=(lane == 1) & in_range)

    pltpu.sync_copy(buf_v, out_hbm.at[pl.ds(base, _CHUNK)])


def kernel(input):
    toks = (
        jnp.zeros((_L,), jnp.int32)
        .at[0].set(input[0, 0])
        .at[1].set(input[1, 0])
    )
    bow = _bow_sc(toks)
    return bow[:_V].reshape(1, _V)


# speedup vs baseline: 1.0775x; 1.0775x over previous
import functools

import jax
import jax.numpy as jnp
from jax import lax
from jax.experimental import pallas as pl
from jax.experimental.pallas import tpu as pltpu
from jax.experimental.pallas import tpu_sc as plsc

_V = 1000
_CHUNK = 32
_L = 16
_NW = 32
_TAIL = _V - (_NW - 1) * _CHUNK

_mesh = plsc.VectorSubcoreMesh(core_axis_name="c", subcore_axis_name="s")


@functools.partial(
    pl.kernel,
    mesh=_mesh,
    out_type=jax.ShapeDtypeStruct((_V,), jnp.float32),
    scratch_types=[
        pltpu.VMEM((_L,), jnp.int32),
        pltpu.VMEM((_CHUNK,), jnp.float32),
    ],
    compiler_params=pltpu.CompilerParams(needs_layout_passes=False),
)
def _bow_sc(tok_hbm, out_hbm, tok_v, buf_v):
    wid = lax.axis_index("s") * 2 + lax.axis_index("c")
    base = wid * _CHUNK

    pltpu.sync_copy(tok_hbm, tok_v.at[pl.ds(0, 2)])
    toks = tok_v[...]

    zeros = jnp.zeros((_L,), jnp.float32)
    buf_v[pl.ds(0, _L)] = zeros
    buf_v[pl.ds(_L, _L)] = zeros

    lane = lax.iota(jnp.int32, _L)
    loc = toks - base
    in_range = (loc >= 0) & (loc < _CHUNK)
    loc_c = jnp.clip(loc, 0, _CHUNK - 1)
    ones = jnp.ones((_L,), jnp.float32)

    plsc.addupdate_scatter(buf_v, [loc_c], ones, mask=(lane == 0) & in_range)
    plsc.addupdate_scatter(buf_v, [loc_c], ones, mask=(lane == 1) & in_range)

    @pl.when(wid < _NW - 1)
    def _():
        pltpu.sync_copy(buf_v, out_hbm.at[pl.ds(base, _CHUNK)])

    @pl.when(wid == _NW - 1)
    def _():
        pltpu.sync_copy(buf_v.at[pl.ds(0, _TAIL)], out_hbm.at[pl.ds(base, _TAIL)])


def kernel(input):
    return _bow_sc(input.reshape(2)).reshape(1, _V)


# speedup vs baseline: 1.1786x; 1.0938x over previous
import functools

import jax
import jax.numpy as jnp
from jax import lax
from jax.experimental import pallas as pl
from jax.experimental.pallas import tpu as pltpu
from jax.experimental.pallas import tpu_sc as plsc

_V = 1000
_CHUNK = 64
_L = 16
_NW = 16
_TAIL = _V - (_NW - 1) * _CHUNK

_mesh = plsc.VectorSubcoreMesh(core_axis_name="c", subcore_axis_name="s",
                               num_cores=1)


@functools.partial(
    pl.kernel,
    mesh=_mesh,
    out_type=jax.ShapeDtypeStruct((_V,), jnp.float32),
    scratch_types=[
        pltpu.VMEM((_L,), jnp.int32),
        pltpu.VMEM((_CHUNK,), jnp.float32),
    ],
    compiler_params=pltpu.CompilerParams(needs_layout_passes=False),
)
def _bow_sc(tok_hbm, out_hbm, tok_v, buf_v):
    wid = lax.axis_index("s")
    base = wid * _CHUNK

    pltpu.sync_copy(tok_hbm, tok_v.at[pl.ds(0, 2)])
    toks = tok_v[...]

    zeros = jnp.zeros((_L,), jnp.float32)
    for i in range(_CHUNK // _L):
        buf_v[pl.ds(i * _L, _L)] = zeros

    lane = lax.iota(jnp.int32, _L)
    loc = toks - base
    in_range = (loc >= 0) & (loc < _CHUNK)
    loc_c = jnp.clip(loc, 0, _CHUNK - 1)
    ones = jnp.ones((_L,), jnp.float32)

    plsc.addupdate_scatter(buf_v, [loc_c], ones, mask=(lane == 0) & in_range)
    plsc.addupdate_scatter(buf_v, [loc_c], ones, mask=(lane == 1) & in_range)

    @pl.when(wid < _NW - 1)
    def _():
        pltpu.sync_copy(buf_v, out_hbm.at[pl.ds(base, _CHUNK)])

    @pl.when(wid == _NW - 1)
    def _():
        pltpu.sync_copy(buf_v.at[pl.ds(0, _TAIL)], out_hbm.at[pl.ds(base, _TAIL)])


def kernel(input):
    return _bow_sc(input.reshape(2)).reshape(1, _V)
